# Initial kernel scaffold; baseline (speedup 1.0000x reference)
import jax
import jax.numpy as jnp
from jax.experimental import pallas as pl


def kernel(x, edge_index, adj_vals, W, b):
    raise NotImplementedError("write your pallas kernel here")


# speedup vs baseline: 3.5623x; 3.5623x over previous
import functools

import jax
import jax.numpy as jnp
from jax import lax
from jax.experimental import pallas as pl
from jax.experimental.pallas import tpu as pltpu
from jax.experimental.pallas import tpu_sc as plsc

N = 10000
F = 128
E = 320000


_BK = 400


def _mm_body(w_ref, x_ref, o_ref):
    k = pl.program_id(0)

    @pl.when(k == 0)
    def _():
        o_ref[...] = jnp.zeros_like(o_ref)

    o_ref[...] += lax.dot_general(
        w_ref[...], x_ref[...],
        (((0,), (0,)), ((), ())),
        preferred_element_type=jnp.float32,
    )


def _matmul_wt_x(W, x):
    return pl.pallas_call(
        _mm_body,
        grid=(N // _BK,),
        in_specs=[
            pl.BlockSpec((_BK, N), lambda k: (k, 0)),
            pl.BlockSpec((_BK, F), lambda k: (k, 0)),
        ],
        out_specs=pl.BlockSpec((N, F), lambda k: (0, 0)),
        out_shape=jax.ShapeDtypeStruct((N, F), jnp.float32),
        compiler_params=pltpu.CompilerParams(
            dimension_semantics=("arbitrary",)),
    )(W, x)


_NC = 2
_NS = 16
_NW = _NC * _NS
_EPW = E // _NW
_CK = 80
_NCH = _EPW // _CK
_RPT = 624
_ZR = 16


def _sc_agg_body(newx_hbm, cols_hbm, rows_hbm, vals_hbm, out_hbm,
                 col_v, row_v, val_v, gat_v, zero_v, accum, sem):
    c = lax.axis_index("c")
    s = lax.axis_index("s")
    wid = s * _NC + c

    row0 = pl.multiple_of(s * _RPT, 8)
    nzc = jnp.where(s == _NS - 1, (N - (_NS - 1) * _RPT) // _ZR, _RPT // _ZR)
    zeros16 = jnp.zeros((16,), jnp.float32)
    for r in range(_ZR):
        for f in range(F // 16):
            zero_v[r, pl.ds(f * 16, 16)] = zeros16

    def _zero_step(j, carry):
        pltpu.sync_copy(zero_v, accum.at[pl.ds(row0 + j * _ZR, _ZR)])
        return carry

    lax.fori_loop(0, nzc, _zero_step, 0)
    plsc.subcore_barrier()

    def _edge_chunk(j, carry):
        base = pl.multiple_of(wid * _EPW + j * _CK, 8)
        pltpu.sync_copy(cols_hbm.at[pl.ds(base, _CK)], col_v)
        pltpu.sync_copy(rows_hbm.at[pl.ds(base, _CK)], row_v)
        pltpu.sync_copy(vals_hbm.at[pl.ds(base, _CK)], val_v)
        pltpu.async_copy(newx_hbm.at[col_v], gat_v, sem).wait()
        for g in range(_CK // 16):
            vv = val_v[pl.ds(g * 16, 16)]
            for l in range(16):
                e = g * 16 + l
                bv = jnp.broadcast_to(vv[l], (16,))
                for f in range(F // 16):
                    sl = pl.ds(f * 16, 16)
                    gat_v[e, sl] = gat_v[e, sl] * bv
        pltpu.sync_copy(gat_v, accum.at[row_v], add=True)
        return carry

    lax.fori_loop(0, _NCH, _edge_chunk, 0)
    plsc.subcore_barrier()

    def _wb_step(j, carry):
        sl = pl.ds(row0 + j * _ZR, _ZR)
        pltpu.sync_copy(accum.at[sl], out_hbm.at[c, sl])
        return carry

    lax.fori_loop(0, nzc, _wb_step, 0)


def _sc_aggregate(new_x, cols, rows, vals):
    mesh = plsc.VectorSubcoreMesh(core_axis_name="c", subcore_axis_name="s",
                                  num_cores=_NC, num_subcores=_NS)
    kern = functools.partial(
        pl.kernel,
        mesh=mesh,
        out_type=jax.ShapeDtypeStruct((_NC, N, F), jnp.float32),
        scratch_types=[
            pltpu.VMEM((_CK,), jnp.int32),
            pltpu.VMEM((_CK,), jnp.int32),
            pltpu.VMEM((_CK,), jnp.float32),
            pltpu.VMEM((_CK, F), jnp.float32),
            pltpu.VMEM((_ZR, F), jnp.float32),
            pltpu.VMEM_SHARED((N, F), jnp.float32),
            pltpu.SemaphoreType.DMA,
        ],
    )(_sc_agg_body)
    return kern(new_x, cols, rows, vals)


_BR = 400


def _combine_body(p_ref, b_ref, o_ref):
    o_ref[...] = p_ref[0] + p_ref[1] + b_ref[...]


def _combine(partials, b):
    return pl.pallas_call(
        _combine_body,
        grid=(N // _BR,),
        in_specs=[
            pl.BlockSpec((_NC, _BR, F), lambda i: (0, i, 0)),
            pl.BlockSpec((_BR, F), lambda i: (i, 0)),
        ],
        out_specs=pl.BlockSpec((_BR, F), lambda i: (i, 0)),
        out_shape=jax.ShapeDtypeStruct((N, F), jnp.float32),
    )(partials, b)


def kernel(x, edge_index, adj_vals, W, b):
    new_x = _matmul_wt_x(W, x)
    partials = _sc_aggregate(new_x, edge_index[1], edge_index[0], adj_vals)
    return _combine(partials, b)


# speedup vs baseline: 6.0545x; 1.6996x over previous
import functools

import jax
import jax.numpy as jnp
from jax import lax
from jax.experimental import pallas as pl
from jax.experimental.pallas import tpu as pltpu
from jax.experimental.pallas import tpu_sc as plsc

N = 10000
F = 128
E = 320000


_BK = 400


def _mm_body(w_ref, x_ref, o_ref):
    k = pl.program_id(0)

    @pl.when(k == 0)
    def _():
        o_ref[...] = jnp.zeros_like(o_ref)

    o_ref[...] += lax.dot_general(
        w_ref[...], x_ref[...],
        (((0,), (0,)), ((), ())),
        preferred_element_type=jnp.float32,
    )


def _matmul_wt_x(W, x):
    return pl.pallas_call(
        _mm_body,
        grid=(N // _BK,),
        in_specs=[
            pl.BlockSpec((_BK, N), lambda k: (k, 0)),
            pl.BlockSpec((_BK, F), lambda k: (k, 0)),
        ],
        out_specs=pl.BlockSpec((N, F), lambda k: (0, 0)),
        out_shape=jax.ShapeDtypeStruct((N, F), jnp.float32),
        compiler_params=pltpu.CompilerParams(
            dimension_semantics=("arbitrary",)),
    )(W, x)


_NC = 2
_NS = 16
_NW = _NC * _NS
_EPW = E // _NW
_CK = 80
_NCH = _EPW // _CK
_RPT = 624
_ZR = 16


def _sc_agg_body(newx_hbm, cols_hbm, rows_hbm, vals_hbm, out_hbm,
                 col1_v, val1_v, row_v0, row_v1, gat0, gat1, zero_v, accum,
                 lsem, gsem0, gsem1, rsem0, rsem1, ssem0, ssem1):
    c = lax.axis_index("c")
    s = lax.axis_index("s")
    wid = s * _NC + c
    ebase = pl.multiple_of(wid * _EPW, 8)

    row0 = pl.multiple_of(s * _RPT, 8)
    nzc = jnp.where(s == _NS - 1, (N - (_NS - 1) * _RPT) // _ZR, _RPT // _ZR)
    zeros16 = jnp.zeros((16,), jnp.float32)
    for r in range(_ZR):
        for f in range(F // 16):
            zero_v[r, pl.ds(f * 16, 16)] = zeros16

    cols_cp = pltpu.async_copy(cols_hbm.at[pl.ds(ebase, _EPW)], col1_v, lsem)
    vals_cp = pltpu.async_copy(vals_hbm.at[pl.ds(ebase, _EPW)], val1_v, lsem)

    def _zero_step(j, carry):
        pltpu.sync_copy(zero_v, accum.at[pl.ds(row0 + j * _ZR, _ZR)])
        return carry

    lax.fori_loop(0, nzc, _zero_step, 0)
    cols_cp.wait()
    vals_cp.wait()
    plsc.subcore_barrier()

    def _fire(j, gat, rv, gsem, rsem):
        pltpu.async_copy(rows_hbm.at[pl.ds(ebase + j * _CK, _CK)], rv, rsem)
        pltpu.async_copy(newx_hbm.at[col1_v.at[pl.ds(j * _CK, _CK)]],
                         gat, gsem)

    def _process(j, gat, rv, gsem, rsem, ssem, ogat, orv, ogsem, orsem, ossem):
        pltpu.make_async_copy(rows_hbm.at[pl.ds(ebase + j * _CK, _CK)],
                              rv, rsem).wait()
        pltpu.make_async_copy(newx_hbm.at[col1_v.at[pl.ds(j * _CK, _CK)]],
                              gat, gsem).wait()

        @pl.when(j > 0)
        def _():
            pltpu.make_async_copy(ogat, accum.at[orv], ossem).wait()

        @pl.when(j < _NCH - 1)
        def _():
            _fire(j + 1, ogat, orv, ogsem, orsem)

        @plsc.parallel_loop(0, _CK // 16, 1)
        def _scale(g):
            vv = val1_v[pl.ds(j * _CK + g * 16, 16)]
            for l in range(16):
                bv = jnp.broadcast_to(vv[l], (16,))
                e = g * 16 + l
                for f in range(F // 16):
                    sl = pl.ds(f * 16, 16)
                    gat[e, sl] = gat[e, sl] * bv

        pltpu.async_copy(gat, accum.at[rv], ssem, add=True)

    _fire(0, gat0, row_v0, gsem0, rsem0)

    def _chunk(j, carry):
        @pl.when(j % 2 == 0)
        def _():
            _process(j, gat0, row_v0, gsem0, rsem0, ssem0,
                     gat1, row_v1, gsem1, rsem1, ssem1)

        @pl.when(j % 2 == 1)
        def _():
            _process(j, gat1, row_v1, gsem1, rsem1, ssem1,
                     gat0, row_v0, gsem0, rsem0, ssem0)

        return carry

    lax.fori_loop(0, _NCH, _chunk, 0)
    pltpu.make_async_copy(gat0, accum.at[row_v0], ssem0).wait()
    plsc.subcore_barrier()

    def _wb_step(j, carry):
        sl = pl.ds(row0 + j * _ZR, _ZR)
        pltpu.sync_copy(accum.at[sl], out_hbm.at[c, sl])
        return carry

    lax.fori_loop(0, nzc, _wb_step, 0)


def _sc_aggregate(new_x, cols, rows, vals):
    mesh = plsc.VectorSubcoreMesh(core_axis_name="c", subcore_axis_name="s",
                                  num_cores=_NC, num_subcores=_NS)
    kern = functools.partial(
        pl.kernel,
        mesh=mesh,
        out_type=jax.ShapeDtypeStruct((_NC, N, F), jnp.float32),
        scratch_types=[
            pltpu.VMEM((_EPW,), jnp.int32),
            pltpu.VMEM((_EPW,), jnp.float32),
            pltpu.VMEM((_CK,), jnp.int32),
            pltpu.VMEM((_CK,), jnp.int32),
            pltpu.VMEM((_CK, F), jnp.float32),
            pltpu.VMEM((_CK, F), jnp.float32),
            pltpu.VMEM((_ZR, F), jnp.float32),
            pltpu.VMEM_SHARED((N, F), jnp.float32),
            pltpu.SemaphoreType.DMA,
            pltpu.SemaphoreType.DMA,
            pltpu.SemaphoreType.DMA,
            pltpu.SemaphoreType.DMA,
            pltpu.SemaphoreType.DMA,
            pltpu.SemaphoreType.DMA,
            pltpu.SemaphoreType.DMA,
        ],
    )(_sc_agg_body)
    return kern(new_x, cols, rows, vals)


_BR = 400


def _combine_body(p_ref, b_ref, o_ref):
    o_ref[...] = p_ref[0] + p_ref[1] + b_ref[...]


def _combine(partials, b):
    return pl.pallas_call(
        _combine_body,
        grid=(N // _BR,),
        in_specs=[
            pl.BlockSpec((_NC, _BR, F), lambda i: (0, i, 0)),
            pl.BlockSpec((_BR, F), lambda i: (i, 0)),
        ],
        out_specs=pl.BlockSpec((_BR, F), lambda i: (i, 0)),
        out_shape=jax.ShapeDtypeStruct((N, F), jnp.float32),
    )(partials, b)


def kernel(x, edge_index, adj_vals, W, b):
    new_x = _matmul_wt_x(W, x)
    partials = _sc_aggregate(new_x, edge_index[1], edge_index[0], adj_vals)
    return _combine(partials, b)


# speedup vs baseline: 6.0641x; 1.0016x over previous
import functools

import jax
import jax.numpy as jnp
from jax import lax
from jax.experimental import pallas as pl
from jax.experimental.pallas import tpu as pltpu
from jax.experimental.pallas import tpu_sc as plsc

N = 10000
F = 128
E = 320000


_BK = 400


def _mm_body(w_ref, x_ref, o_ref):
    k = pl.program_id(0)

    @pl.when(k == 0)
    def _():
        o_ref[...] = jnp.zeros_like(o_ref)

    o_ref[...] += lax.dot_general(
        w_ref[...], x_ref[...],
        (((0,), (0,)), ((), ())),
        preferred_element_type=jnp.float32,
    )


def _matmul_wt_x(W, x):
    return pl.pallas_call(
        _mm_body,
        grid=(N // _BK,),
        in_specs=[
            pl.BlockSpec((_BK, N), lambda k: (k, 0)),
            pl.BlockSpec((_BK, F), lambda k: (k, 0)),
        ],
        out_specs=pl.BlockSpec((N, F), lambda k: (0, 0)),
        out_shape=jax.ShapeDtypeStruct((N, F), jnp.float32),
        compiler_params=pltpu.CompilerParams(
            dimension_semantics=("arbitrary",)),
    )(W, x)


_NC = 2
_NS = 16
_NW = _NC * _NS
_EPW = E // _NW
_CK = 80
_NCH = _EPW // _CK
_RPT = 624
_ZR = 16


def _sc_agg_body(newx_hbm, cols_hbm, rows_hbm, vals_hbm, out_hbm,
                 col_v0, col_v1, col_v2, col_v3,
                 row_v0, row_v1, row_v2, row_v3,
                 val_v0, val_v1, val_v2, val_v3,
                 gat0, gat1, gat2, gat3, zero_v, accum,
                 csem0, csem1, csem2, csem3,
                 rsem0, rsem1, rsem2, rsem3,
                 vsem0, vsem1, vsem2, vsem3,
                 gsem0, gsem1, gsem2, gsem3,
                 ssem0, ssem1, ssem2, ssem3):
    c = lax.axis_index("c")
    s = lax.axis_index("s")
    wid = s * _NC + c
    ebase = pl.multiple_of(wid * _EPW, 8)

    row0 = pl.multiple_of(s * _RPT, 8)
    nzc = jnp.where(s == _NS - 1, (N - (_NS - 1) * _RPT) // _ZR, _RPT // _ZR)
    zeros16 = jnp.zeros((16,), jnp.float32)
    for r in range(_ZR):
        for f in range(F // 16):
            zero_v[r, pl.ds(f * 16, 16)] = zeros16

    def _zero_step(j, carry):
        pltpu.sync_copy(zero_v, accum.at[pl.ds(row0 + j * _ZR, _ZR)])
        return carry

    lax.fori_loop(0, nzc, _zero_step, 0)
    plsc.subcore_barrier()

    def _fire_idx(j, cv, rv, vv, csem, rsem, vsem):
        sl = pl.ds(ebase + j * _CK, _CK)
        pltpu.async_copy(cols_hbm.at[sl], cv, csem)
        pltpu.async_copy(rows_hbm.at[sl], rv, rsem)
        pltpu.async_copy(vals_hbm.at[sl], vv, vsem)

    def _wait_idx(j, cv, rv, vv, csem, rsem, vsem):
        sl = pl.ds(ebase + j * _CK, _CK)
        pltpu.make_async_copy(cols_hbm.at[sl], cv, csem).wait()
        pltpu.make_async_copy(rows_hbm.at[sl], rv, rsem).wait()
        pltpu.make_async_copy(vals_hbm.at[sl], vv, vsem).wait()

    def _process(j, cur, pre, dr):
        (gat, cv, rv, vv, csem, rsem, vsem, gsem, ssem) = cur
        (pgat, pcv, prv, pvv, pcsem, prsem, pvsem, pgsem, pssem) = pre
        (dgat, dcv, drv, dvv, dcsem, drsem, dvsem, dgsem, dssem) = dr

        @pl.when(j > 1)
        def _():
            pltpu.make_async_copy(dgat, accum.at[drv], dssem).wait()

        @pl.when(j < _NCH - 2)
        def _():
            _fire_idx(j + 2, dcv, drv, dvv, dcsem, drsem, dvsem)

        @pl.when(j < _NCH - 1)
        def _():
            _wait_idx(j + 1, pcv, prv, pvv, pcsem, prsem, pvsem)
            pltpu.async_copy(newx_hbm.at[pcv], pgat, pgsem)

        pltpu.make_async_copy(newx_hbm.at[cv], gat, gsem).wait()

        @plsc.parallel_loop(0, _CK // 16, 1)
        def _scale(g):
            vvv = vv[pl.ds(g * 16, 16)]
            for l in range(16):
                bv = jnp.broadcast_to(vvv[l], (16,))
                e = g * 16 + l
                for f in range(F // 16):
                    sl = pl.ds(f * 16, 16)
                    gat[e, sl] = gat[e, sl] * bv

        pltpu.async_copy(gat, accum.at[rv], ssem, add=True)

    bufs = ((gat0, col_v0, row_v0, val_v0, csem0, rsem0, vsem0, gsem0, ssem0),
            (gat1, col_v1, row_v1, val_v1, csem1, rsem1, vsem1, gsem1, ssem1),
            (gat2, col_v2, row_v2, val_v2, csem2, rsem2, vsem2, gsem2, ssem2),
            (gat3, col_v3, row_v3, val_v3, csem3, rsem3, vsem3, gsem3, ssem3))
    _fire_idx(0, bufs[0][1], bufs[0][2], bufs[0][3], *bufs[0][4:7])
    _fire_idx(1, bufs[1][1], bufs[1][2], bufs[1][3], *bufs[1][4:7])
    _wait_idx(0, bufs[0][1], bufs[0][2], bufs[0][3], *bufs[0][4:7])
    pltpu.async_copy(newx_hbm.at[bufs[0][1]], bufs[0][0], bufs[0][7])

    def _chunk(j, carry):
        for p in range(4):
            @pl.when(j % 4 == p)
            def _(p=p):
                _process(j, bufs[p], bufs[(p + 1) % 4], bufs[(p + 2) % 4])

        return carry

    lax.fori_loop(0, _NCH, _chunk, 0)
    for jt in (_NCH - 2, _NCH - 1):
        p = jt % 4
        pltpu.make_async_copy(bufs[p][0], accum.at[bufs[p][2]],
                              bufs[p][8]).wait()
    plsc.subcore_barrier()

    def _wb_step(j, carry):
        sl = pl.ds(row0 + j * _ZR, _ZR)
        pltpu.sync_copy(accum.at[sl], out_hbm.at[c, sl])
        return carry

    lax.fori_loop(0, nzc, _wb_step, 0)


def _sc_aggregate(new_x, cols, rows, vals):
    mesh = plsc.VectorSubcoreMesh(core_axis_name="c", subcore_axis_name="s",
                                  num_cores=_NC, num_subcores=_NS)
    kern = functools.partial(
        pl.kernel,
        mesh=mesh,
        out_type=jax.ShapeDtypeStruct((_NC, N, F), jnp.float32),
        scratch_types=(
            [pltpu.VMEM((_CK,), jnp.int32)] * 4
            + [pltpu.VMEM((_CK,), jnp.int32)] * 4
            + [pltpu.VMEM((_CK,), jnp.float32)] * 4
            + [pltpu.VMEM((_CK, F), jnp.float32)] * 4
            + [pltpu.VMEM((_ZR, F), jnp.float32),
               pltpu.VMEM_SHARED((N, F), jnp.float32)]
            + [pltpu.SemaphoreType.DMA] * 20
        ),
    )(_sc_agg_body)
    return kern(new_x, cols, rows, vals)


_BR = 400


def _combine_body(p_ref, b_ref, o_ref):
    o_ref[...] = p_ref[0] + p_ref[1] + b_ref[...]


def _combine(partials, b):
    return pl.pallas_call(
        _combine_body,
        grid=(N // _BR,),
        in_specs=[
            pl.BlockSpec((_NC, _BR, F), lambda i: (0, i, 0)),
            pl.BlockSpec((_BR, F), lambda i: (i, 0)),
        ],
        out_specs=pl.BlockSpec((_BR, F), lambda i: (i, 0)),
        out_shape=jax.ShapeDtypeStruct((N, F), jnp.float32),
    )(partials, b)


def kernel(x, edge_index, adj_vals, W, b):
    new_x = _matmul_wt_x(W, x)
    partials = _sc_aggregate(new_x, edge_index[1], edge_index[0], adj_vals)
    return _combine(partials, b)
